# baseline (speedup 1.0000x reference)
import functools

import jax
import jax.numpy as jnp
from jax import lax
from jax.experimental import pallas as pl
from jax.experimental.pallas import tpu as pltpu
from jax.experimental.pallas import tpu_sc as plsc

N = 1024
D_C = 512
D_K = 256
D_T = 512
N_KB = 100000
TOPK = 32

R = 1024
CK = 4096
NCH = 25
NKB_PAD = NCH * CK
NLT = CK // 128
IMIN = -2 ** 31
IDXM = 131071


def _topk_kernel(cs_ref, qw_ref, keys_ref, idx_out_ref, q_out_ref,
                 q16_ref, s_ref, top_ref, pm_ref, done_ref):
    j = pl.program_id(1)

    @pl.when(j == 0)
    def _init():
        qf = lax.dot_general(
            cs_ref[...], qw_ref[...], (((1,), (1,)), ((), ())),
            preferred_element_type=jnp.float32)
        q_out_ref[...] = qf
        q16_ref[...] = qf.astype(jnp.bfloat16)
        top_ref[...] = jnp.full((R, TOPK), IMIN, jnp.int32)

    s = lax.dot_general(
        q16_ref[...], keys_ref[...], (((1,), (1,)), ((), ())),
        preferred_element_type=jnp.float32)
    bits = lax.bitcast_convert_type(s, jnp.int32)
    key = bits ^ (lax.shift_right_arithmetic(bits, 31) & (2 ** 31 - 1))
    gidx = j * CK + lax.broadcasted_iota(jnp.int32, (R, CK), 1)
    packed = (key & ~IDXM) | (IDXM - gidx)
    s_ref[...] = jnp.where(gidx < N_KB, packed, IMIN)
    pm_ref[...] = jnp.full((R, 1), 2 ** 31 - 1, jnp.int32)
    done_ref[0] = 0

    lane = lax.broadcasted_iota(jnp.int32, (R, TOPK), 1)

    def _extract(_, carry):
        @pl.when(done_ref[0] == 0)
        def _():
            p = s_ref[...]
            prev = pm_ref[...]
            pm = jnp.max(jnp.where(p < prev, p, IMIN),
                         axis=1, keepdims=True)
            t32 = top_ref[...]
            min32 = jnp.min(t32, axis=1, keepdims=True)
            need = pm > min32
            any_need = jnp.max(jnp.where(need, 1, 0)) > 0

            @pl.when(jnp.logical_not(any_need))
            def _():
                done_ref[0] = 1

            @pl.when(any_need)
            def _():
                pm_ref[...] = pm
                pos = jnp.min(jnp.where(t32 == min32, lane, 2 ** 30),
                              axis=1, keepdims=True)
                sel = (lane == pos) & need
                top_ref[...] = jnp.where(sel, pm, t32)
        return carry

    lax.fori_loop(0, TOPK, _extract, 0)

    @pl.when(j == NCH - 1)
    def _finish():
        idx_out_ref[...] = IDXM - (top_ref[...] & IDXM)


@jax.jit
def _topk_call(cs, qw, keys16):
    return pl.pallas_call(
        _topk_kernel,
        grid=(N // R, NCH),
        in_specs=[
            pl.BlockSpec((R, D_C), lambda i, j: (i, 0)),
            pl.BlockSpec((D_T, D_C), lambda i, j: (0, 0)),
            pl.BlockSpec((CK, D_T), lambda i, j: (j, 0)),
        ],
        out_specs=[
            pl.BlockSpec((R, TOPK), lambda i, j: (i, 0)),
            pl.BlockSpec((R, D_T), lambda i, j: (i, 0)),
        ],
        out_shape=[
            jax.ShapeDtypeStruct((N, TOPK), jnp.int32),
            jax.ShapeDtypeStruct((N, D_T), jnp.float32),
        ],
        scratch_shapes=[
            pltpu.VMEM((R, D_T), jnp.bfloat16),
            pltpu.VMEM((R, CK), jnp.int32),
            pltpu.VMEM((R, TOPK), jnp.int32),
            pltpu.VMEM((R, 1), jnp.int32),
            pltpu.SMEM((1,), jnp.int32),
        ],
        compiler_params=pltpu.CompilerParams(
            dimension_semantics=("arbitrary", "arbitrary")),
    )(cs, qw, keys16)


_NW = 32
_B = N * TOPK
_BPW = _B // _NW
_GCH = 128
_NGC = _BPW // _GCH


def _gather_body(idx_hbm, table_hbm, out_hbm, idx_v, rows_v, sem):
    wid = lax.axis_index("s") * 2 + lax.axis_index("c")
    base = wid * _BPW

    def body(c, carry):
        off = base + c * _GCH
        pltpu.sync_copy(idx_hbm.at[pl.ds(off, _GCH)], idx_v)
        pltpu.async_copy(table_hbm.at[idx_v], rows_v, sem).wait()
        pltpu.sync_copy(rows_v, out_hbm.at[pl.ds(off, _GCH)])
        return carry

    lax.fori_loop(0, _NGC, body, 0)


@jax.jit
def _gather_call(idx_flat, table):
    f = functools.partial(
        pl.kernel,
        mesh=plsc.VectorSubcoreMesh(core_axis_name="c", subcore_axis_name="s"),
        out_type=jax.ShapeDtypeStruct((_B, D_T), jnp.float32),
        scratch_types=[
            pltpu.VMEM((_GCH,), jnp.int32),
            pltpu.VMEM((_GCH, D_T), jnp.float32),
            pltpu.SemaphoreType.DMA,
        ],
    )(_gather_body)
    return f(idx_flat, table)


RB = 32


def _fuse_kernel(q_ref, c_ref, k_ref, gk_ref, gv_ref, o_ref):
    q = q_ref[...]
    sks = []
    for kk in range(TOPK):
        sks.append(jnp.sum(q * gk_ref[:, kk, :], axis=1, keepdims=True))
    s = jnp.concatenate(sks, axis=1)
    mx = jnp.max(s, axis=1, keepdims=True)
    e = jnp.exp(s - mx)
    w = e / jnp.sum(e, axis=1, keepdims=True)
    acc = jnp.zeros((RB, D_T), jnp.float32)
    for kk in range(TOPK):
        acc = acc + w[:, kk:kk + 1] * gv_ref[:, kk, :]
    o_ref[...] = jnp.concatenate([
        jnp.maximum(c_ref[...], 0.0),
        jnp.maximum(k_ref[...], 0.0),
        jnp.maximum(0.5 * acc, 0.0),
    ], axis=1)


@jax.jit
def _fuse_call(q, C, K, gk, gv):
    return pl.pallas_call(
        _fuse_kernel,
        grid=(N // RB,),
        in_specs=[
            pl.BlockSpec((RB, D_T), lambda i: (i, 0)),
            pl.BlockSpec((RB, D_C), lambda i: (i, 0)),
            pl.BlockSpec((RB, D_K), lambda i: (i, 0)),
            pl.BlockSpec((RB, TOPK, D_T), lambda i: (i, 0, 0)),
            pl.BlockSpec((RB, TOPK, D_T), lambda i: (i, 0, 0)),
        ],
        out_specs=pl.BlockSpec((RB, D_C + D_K + D_T), lambda i: (i, 0)),
        out_shape=jax.ShapeDtypeStruct((N, D_C + D_K + D_T), jnp.float32),
    )(q, C, K, gk, gv)


def kernel(C, K, tie_kb_keys, tie_kb_values, Q_w, top_k, temperature):
    del top_k
    cs = C / temperature
    keys16 = jnp.pad(tie_kb_keys, ((0, NKB_PAD - N_KB), (0, 0))
                     ).astype(jnp.bfloat16)
    idx, q = _topk_call(cs, Q_w, keys16)
    idx_flat = idx.reshape(-1)
    gv = _gather_call(idx_flat, tie_kb_values).reshape(N, TOPK, D_T)
    gk = _gather_call(idx_flat, tie_kb_keys).reshape(N, TOPK, D_T)
    return _fuse_call(q, C, K, gk, gv)


# speedup vs baseline: 1.0087x; 1.0087x over previous
import functools

import jax
import jax.numpy as jnp
from jax import lax
from jax.experimental import pallas as pl
from jax.experimental.pallas import tpu as pltpu
from jax.experimental.pallas import tpu_sc as plsc

N = 1024
D_C = 512
D_K = 256
D_T = 512
N_KB = 100000
TOPK = 32

R = 1024
CK = 2048
NCH = 49
NKB_PAD = NCH * CK
NLT = CK // 128
IMIN = -2 ** 31
IDXM = 131071


def _topk_kernel(cs_ref, qw_ref, keys_ref, idx_out_ref, q_out_ref,
                 q16_ref, s_ref, top_ref, pm_ref, done_ref):
    j = pl.program_id(1)

    @pl.when(j == 0)
    def _init():
        qf = lax.dot_general(
            cs_ref[...], qw_ref[...], (((1,), (1,)), ((), ())),
            preferred_element_type=jnp.float32)
        q_out_ref[...] = qf
        q16_ref[...] = qf.astype(jnp.bfloat16)
        top_ref[...] = jnp.full((R, TOPK), IMIN, jnp.int32)

    s = lax.dot_general(
        q16_ref[...], keys_ref[...], (((1,), (1,)), ((), ())),
        preferred_element_type=jnp.float32)
    bits = lax.bitcast_convert_type(s, jnp.int32)
    key = bits ^ (lax.shift_right_arithmetic(bits, 31) & (2 ** 31 - 1))
    gidx = j * CK + lax.broadcasted_iota(jnp.int32, (R, CK), 1)
    packed = (key & ~IDXM) | (IDXM - gidx)
    s_ref[...] = jnp.where(gidx < N_KB, packed, IMIN)
    pm_ref[...] = jnp.full((R, 1), 2 ** 31 - 1, jnp.int32)
    done_ref[0] = 0

    lane = lax.broadcasted_iota(jnp.int32, (R, TOPK), 1)

    def _extract(_, carry):
        @pl.when(done_ref[0] == 0)
        def _():
            p = s_ref[...]
            prev = pm_ref[...]
            pm = jnp.max(jnp.where(p < prev, p, IMIN),
                         axis=1, keepdims=True)
            t32 = top_ref[...]
            min32 = jnp.min(t32, axis=1, keepdims=True)
            need = pm > min32
            any_need = jnp.max(jnp.where(need, 1, 0)) > 0

            @pl.when(jnp.logical_not(any_need))
            def _():
                done_ref[0] = 1

            @pl.when(any_need)
            def _():
                pm_ref[...] = pm
                pos = jnp.min(jnp.where(t32 == min32, lane, 2 ** 30),
                              axis=1, keepdims=True)
                sel = (lane == pos) & need
                top_ref[...] = jnp.where(sel, pm, t32)
        return carry

    lax.fori_loop(0, TOPK, _extract, 0)

    @pl.when(j == NCH - 1)
    def _finish():
        idx_out_ref[...] = IDXM - (top_ref[...] & IDXM)


@jax.jit
def _topk_call(cs, qw, keys16):
    return pl.pallas_call(
        _topk_kernel,
        grid=(N // R, NCH),
        in_specs=[
            pl.BlockSpec((R, D_C), lambda i, j: (i, 0)),
            pl.BlockSpec((D_T, D_C), lambda i, j: (0, 0)),
            pl.BlockSpec((CK, D_T), lambda i, j: (j, 0)),
        ],
        out_specs=[
            pl.BlockSpec((R, TOPK), lambda i, j: (i, 0)),
            pl.BlockSpec((R, D_T), lambda i, j: (i, 0)),
        ],
        out_shape=[
            jax.ShapeDtypeStruct((N, TOPK), jnp.int32),
            jax.ShapeDtypeStruct((N, D_T), jnp.float32),
        ],
        scratch_shapes=[
            pltpu.VMEM((R, D_T), jnp.bfloat16),
            pltpu.VMEM((R, CK), jnp.int32),
            pltpu.VMEM((R, TOPK), jnp.int32),
            pltpu.VMEM((R, 1), jnp.int32),
            pltpu.SMEM((1,), jnp.int32),
        ],
        compiler_params=pltpu.CompilerParams(
            dimension_semantics=("arbitrary", "arbitrary")),
    )(cs, qw, keys16)


_NW = 32
_B = N * TOPK
_BPW = _B // _NW
_GCH = 128
_NGC = _BPW // _GCH


def _gather_body(idx_hbm, table_hbm, out_hbm, idx_v, rows_v, sem):
    wid = lax.axis_index("s") * 2 + lax.axis_index("c")
    base = wid * _BPW

    def body(c, carry):
        off = base + c * _GCH
        pltpu.sync_copy(idx_hbm.at[pl.ds(off, _GCH)], idx_v)
        pltpu.async_copy(table_hbm.at[idx_v], rows_v, sem).wait()
        pltpu.sync_copy(rows_v, out_hbm.at[pl.ds(off, _GCH)])
        return carry

    lax.fori_loop(0, _NGC, body, 0)


@jax.jit
def _gather_call(idx_flat, table):
    f = functools.partial(
        pl.kernel,
        mesh=plsc.VectorSubcoreMesh(core_axis_name="c", subcore_axis_name="s"),
        out_type=jax.ShapeDtypeStruct((_B, D_T), jnp.float32),
        scratch_types=[
            pltpu.VMEM((_GCH,), jnp.int32),
            pltpu.VMEM((_GCH, D_T), jnp.float32),
            pltpu.SemaphoreType.DMA,
        ],
    )(_gather_body)
    return f(idx_flat, table)


RB = 32


def _fuse_kernel(q_ref, c_ref, k_ref, gk_ref, gv_ref, o_ref):
    q = q_ref[...]
    sks = []
    for kk in range(TOPK):
        sks.append(jnp.sum(q * gk_ref[:, kk, :], axis=1, keepdims=True))
    s = jnp.concatenate(sks, axis=1)
    mx = jnp.max(s, axis=1, keepdims=True)
    e = jnp.exp(s - mx)
    w = e / jnp.sum(e, axis=1, keepdims=True)
    acc = jnp.zeros((RB, D_T), jnp.float32)
    for kk in range(TOPK):
        acc = acc + w[:, kk:kk + 1] * gv_ref[:, kk, :]
    o_ref[...] = jnp.concatenate([
        jnp.maximum(c_ref[...], 0.0),
        jnp.maximum(k_ref[...], 0.0),
        jnp.maximum(0.5 * acc, 0.0),
    ], axis=1)


@jax.jit
def _fuse_call(q, C, K, gk, gv):
    return pl.pallas_call(
        _fuse_kernel,
        grid=(N // RB,),
        in_specs=[
            pl.BlockSpec((RB, D_T), lambda i: (i, 0)),
            pl.BlockSpec((RB, D_C), lambda i: (i, 0)),
            pl.BlockSpec((RB, D_K), lambda i: (i, 0)),
            pl.BlockSpec((RB, TOPK, D_T), lambda i: (i, 0, 0)),
            pl.BlockSpec((RB, TOPK, D_T), lambda i: (i, 0, 0)),
        ],
        out_specs=pl.BlockSpec((RB, D_C + D_K + D_T), lambda i: (i, 0)),
        out_shape=jax.ShapeDtypeStruct((N, D_C + D_K + D_T), jnp.float32),
    )(q, C, K, gk, gv)


def kernel(C, K, tie_kb_keys, tie_kb_values, Q_w, top_k, temperature):
    del top_k
    cs = C / temperature
    keys16 = jnp.pad(tie_kb_keys, ((0, NKB_PAD - N_KB), (0, 0))
                     ).astype(jnp.bfloat16)
    idx, q = _topk_call(cs, Q_w, keys16)
    idx_flat = idx.reshape(-1)
    gv = _gather_call(idx_flat, tie_kb_values).reshape(N, TOPK, D_T)
    gk = _gather_call(idx_flat, tie_kb_keys).reshape(N, TOPK, D_T)
    return _fuse_call(q, C, K, gk, gv)


# speedup vs baseline: 1.0183x; 1.0095x over previous
import functools

import jax
import jax.numpy as jnp
from jax import lax
from jax.experimental import pallas as pl
from jax.experimental.pallas import tpu as pltpu
from jax.experimental.pallas import tpu_sc as plsc

N = 1024
D_C = 512
D_K = 256
D_T = 512
N_KB = 100000
TOPK = 32

R = 1024
CK = 2048
NCH = 49
NKB_PAD = NCH * CK
NLT = CK // 128
IMIN = -2 ** 31
IDXM = 131071


def _topk_kernel(cs_ref, qw_ref, keys_ref, idx_out_ref, q_out_ref,
                 q16_ref, s_ref, top_ref, pm_ref, done_ref):
    j = pl.program_id(1)

    @pl.when(j == 0)
    def _init():
        qf = lax.dot_general(
            cs_ref[...], qw_ref[...], (((1,), (1,)), ((), ())),
            preferred_element_type=jnp.float32)
        q_out_ref[...] = qf
        q16_ref[...] = qf.astype(jnp.bfloat16)
        top_ref[...] = jnp.full((R, TOPK), IMIN, jnp.int32)

    s = lax.dot_general(
        q16_ref[...], keys_ref[...], (((1,), (1,)), ((), ())),
        preferred_element_type=jnp.float32)
    bits = lax.bitcast_convert_type(s, jnp.int32)
    key = bits ^ (lax.shift_right_arithmetic(bits, 31) & (2 ** 31 - 1))
    gidx = j * CK + lax.broadcasted_iota(jnp.int32, (R, CK), 1)
    packed = (key & ~IDXM) | (IDXM - gidx)
    s_ref[...] = jnp.where(gidx < N_KB, packed, IMIN)
    pm_ref[...] = jnp.full((R, 1), 2 ** 31 - 1, jnp.int32)
    done_ref[0] = 0

    lane = lax.broadcasted_iota(jnp.int32, (R, TOPK), 1)

    def _extract(_, carry):
        @pl.when(done_ref[0] == 0)
        def _():
            p = s_ref[...]
            prev = pm_ref[...]
            pm = jnp.max(jnp.where(p < prev, p, IMIN),
                         axis=1, keepdims=True)
            t32 = top_ref[...]
            min32 = jnp.min(t32, axis=1, keepdims=True)
            need = pm > min32
            any_need = jnp.max(jnp.where(need, 1, 0)) > 0

            @pl.when(jnp.logical_not(any_need))
            def _():
                done_ref[0] = 1

            @pl.when(any_need)
            def _():
                pm_ref[...] = pm
                pos = jnp.min(jnp.where(t32 == min32, lane, 2 ** 30),
                              axis=1, keepdims=True)
                sel = (lane == pos) & need
                top_ref[...] = jnp.where(sel, pm, t32)
        return carry

    lax.fori_loop(0, TOPK, _extract, 0)

    @pl.when(j == NCH - 1)
    def _finish():
        idx_out_ref[...] = IDXM - (top_ref[...] & IDXM)


@jax.jit
def _topk_call(cs, qw, keys16):
    return pl.pallas_call(
        _topk_kernel,
        grid=(N // R, NCH),
        in_specs=[
            pl.BlockSpec((R, D_C), lambda i, j: (i, 0)),
            pl.BlockSpec((D_T, D_C), lambda i, j: (0, 0)),
            pl.BlockSpec((CK, D_T), lambda i, j: (j, 0)),
        ],
        out_specs=[
            pl.BlockSpec((R, TOPK), lambda i, j: (i, 0)),
            pl.BlockSpec((R, D_T), lambda i, j: (i, 0)),
        ],
        out_shape=[
            jax.ShapeDtypeStruct((N, TOPK), jnp.int32),
            jax.ShapeDtypeStruct((N, D_T), jnp.float32),
        ],
        scratch_shapes=[
            pltpu.VMEM((R, D_T), jnp.bfloat16),
            pltpu.VMEM((R, CK), jnp.int32),
            pltpu.VMEM((R, TOPK), jnp.int32),
            pltpu.VMEM((R, 1), jnp.int32),
            pltpu.SMEM((1,), jnp.int32),
        ],
        compiler_params=pltpu.CompilerParams(
            dimension_semantics=("arbitrary", "arbitrary")),
    )(cs, qw, keys16)


_NW = 32
_B = N * TOPK
_BPW = _B // _NW
_GCH = 128
_NGC = _BPW // _GCH


def _gather_body(idx_hbm, values_hbm, keys_hbm, gv_hbm, gk_hbm,
                 idx_v, rows_v, sem):
    wid = lax.axis_index("s") * 2 + lax.axis_index("c")
    base = wid * _BPW

    def body(c, carry):
        off = base + c * _GCH
        pltpu.sync_copy(idx_hbm.at[pl.ds(off, _GCH)], idx_v)
        pltpu.async_copy(values_hbm.at[idx_v], rows_v, sem).wait()
        pltpu.sync_copy(rows_v, gv_hbm.at[pl.ds(off, _GCH)])
        pltpu.async_copy(keys_hbm.at[idx_v], rows_v, sem).wait()
        pltpu.sync_copy(rows_v, gk_hbm.at[pl.ds(off, _GCH)])
        return carry

    lax.fori_loop(0, _NGC, body, 0)


@jax.jit
def _gather_call(idx_flat, values, keys):
    f = functools.partial(
        pl.kernel,
        mesh=plsc.VectorSubcoreMesh(core_axis_name="c", subcore_axis_name="s"),
        out_type=[
            jax.ShapeDtypeStruct((_B, D_T), jnp.float32),
            jax.ShapeDtypeStruct((_B, D_T), jnp.float32),
        ],
        scratch_types=[
            pltpu.VMEM((_GCH,), jnp.int32),
            pltpu.VMEM((_GCH, D_T), jnp.float32),
            pltpu.SemaphoreType.DMA,
        ],
    )(_gather_body)
    return f(idx_flat, values, keys)


RB = 64


def _fuse_kernel(q_ref, c_ref, k_ref, gk_ref, gv_ref, o_ref):
    q = q_ref[...]
    sks = []
    for kk in range(TOPK):
        sks.append(jnp.sum(q * gk_ref[:, kk, :], axis=1, keepdims=True))
    s = jnp.concatenate(sks, axis=1)
    mx = jnp.max(s, axis=1, keepdims=True)
    e = jnp.exp(s - mx)
    w = e / jnp.sum(e, axis=1, keepdims=True)
    acc = jnp.zeros((RB, D_T), jnp.float32)
    for kk in range(TOPK):
        acc = acc + w[:, kk:kk + 1] * gv_ref[:, kk, :]
    o_ref[...] = jnp.concatenate([
        jnp.maximum(c_ref[...], 0.0),
        jnp.maximum(k_ref[...], 0.0),
        jnp.maximum(0.5 * acc, 0.0),
    ], axis=1)


@jax.jit
def _fuse_call(q, C, K, gk, gv):
    return pl.pallas_call(
        _fuse_kernel,
        grid=(N // RB,),
        in_specs=[
            pl.BlockSpec((RB, D_T), lambda i: (i, 0)),
            pl.BlockSpec((RB, D_C), lambda i: (i, 0)),
            pl.BlockSpec((RB, D_K), lambda i: (i, 0)),
            pl.BlockSpec((RB, TOPK, D_T), lambda i: (i, 0, 0)),
            pl.BlockSpec((RB, TOPK, D_T), lambda i: (i, 0, 0)),
        ],
        out_specs=pl.BlockSpec((RB, D_C + D_K + D_T), lambda i: (i, 0)),
        out_shape=jax.ShapeDtypeStruct((N, D_C + D_K + D_T), jnp.float32),
    )(q, C, K, gk, gv)


def kernel(C, K, tie_kb_keys, tie_kb_values, Q_w, top_k, temperature):
    del top_k
    cs = C / temperature
    keys16 = jnp.pad(tie_kb_keys, ((0, NKB_PAD - N_KB), (0, 0))
                     ).astype(jnp.bfloat16)
    idx, q = _topk_call(cs, Q_w, keys16)
    gv, gk = _gather_call(idx.reshape(-1), tie_kb_values, tie_kb_keys)
    return _fuse_call(q, C, K, gk.reshape(N, TOPK, D_T),
                      gv.reshape(N, TOPK, D_T))
